# baseline (speedup 1.0000x reference)
import functools

import jax
import jax.numpy as jnp
from jax import lax
from jax.experimental import pallas as pl
from jax.experimental.pallas import tpu as pltpu
from jax.experimental.pallas import tpu_sc as plsc

B, P, D, K = 2, 4096, 768, 16
NC, NS, L = 2, 16, 16
NW = NC * NS
ROWS = B * P
RPW = ROWS // NW
G = 4
NBUF = 2
NG = RPW // G
L2 = 2 * L
DW = D // 2
DV = DW // L
DU = 4

_mesh = plsc.VectorSubcoreMesh(
    core_axis_name="c", subcore_axis_name="s", num_cores=NC, num_subcores=NS
)


@functools.partial(
    pl.kernel,
    out_type=jax.ShapeDtypeStruct((ROWS, DW), jnp.uint32),
    mesh=_mesh,
    compiler_params=pltpu.CompilerParams(needs_layout_passes=False),
    scratch_types=[
        [pltpu.VMEM((G * K,), jnp.int32) for _ in range(NBUF)],
        [pltpu.VMEM((G * K,), jnp.float32) for _ in range(NBUF)],
        [pltpu.VMEM((G * K, DW), jnp.uint32) for _ in range(NBUF)],
        pltpu.VMEM((G, DW), jnp.uint32),
        [pltpu.SemaphoreType.DMA for _ in range(NBUF)],
    ],
)
def _wormhole_gather(x_hbm, routes_hbm, weights_hbm, out_hbm,
                     idx_bufs, w_bufs, row_bufs, out_v, sems):
    wid = lax.axis_index("s") * NC + lax.axis_index("c")
    wstart = wid * RPW
    boff = (wid // (NW // B)) * P
    off_splat = jnp.broadcast_to(boff, (L,)).astype(jnp.int32)

    def issue(c, slot):
        base_k = pl.multiple_of((wstart + c * G) * K, G * K)
        pltpu.sync_copy(routes_hbm.at[pl.ds(base_k, G * K)], idx_bufs[slot])
        for i in range(G * K // L):
            sl = pl.ds(i * L, L)
            idx_bufs[slot][sl] = idx_bufs[slot][sl] + off_splat
        pltpu.sync_copy(weights_hbm.at[pl.ds(base_k, G * K)], w_bufs[slot])
        pltpu.make_async_copy(
            x_hbm.at[idx_bufs[slot]], row_bufs[slot], sems[slot]
        ).start()

    def compute(c, slot):
        base = wstart + c * G
        pltpu.make_async_copy(
            x_hbm.at[idx_bufs[slot]], row_bufs[slot], sems[slot]
        ).wait()
        rows = row_bufs[slot]
        for g in range(G):
            wv = w_bufs[slot][pl.ds(g * K, K)]
            u = plsc.bitcast(wv, jnp.uint32)
            r = (u + 0x7FFF + ((u >> 16) & 1)) >> 16
            pk = r | (r << 16)
            ws = [
                plsc.bitcast(jnp.broadcast_to(pk[k], (L,)), jnp.bfloat16)
                for k in range(K)
            ]

            def dbody(d, carry, g=g, ws=ws):
                for u_ in range(DU):
                    sl = pl.ds((d * DU + u_) * L, L)
                    ld = lambda k: plsc.bitcast(rows[g * K + k, sl],
                                                jnp.bfloat16)
                    acc = [ld(a) * ws[a] for a in range(4)]
                    for k in range(4, K):
                        acc[k % 4] = acc[k % 4] + ld(k) * ws[k]
                    s = (acc[0] + acc[1]) + (acc[2] + acc[3])
                    out_v[g, sl] = plsc.bitcast(s, jnp.uint32)
                return carry

            lax.fori_loop(0, DV // DU, dbody, 0)
        pltpu.sync_copy(out_v, out_hbm.at[pl.ds(base, G)])

    issue(0, 0)

    def outer(cc, carry):
        for bslot in range(NBUF):
            c = cc * NBUF + bslot
            nxt_slot = (bslot + 1) % NBUF

            @pl.when(c + 1 < NG)
            def _issue_next():
                issue(c + 1, nxt_slot)

            compute(c, bslot)
        return carry

    lax.fori_loop(0, NG // NBUF, outer, 0)


def kernel(x, routes, weights):
    x_bf = x.astype(jnp.bfloat16).reshape(ROWS, DW, 2)
    x_u32 = lax.bitcast_convert_type(x_bf, jnp.uint32)
    r_flat = routes.astype(jnp.int32).reshape(ROWS * K)
    w_flat = weights.reshape(ROWS * K)
    out_u32 = _wormhole_gather(x_u32, r_flat, w_flat)
    out_bf = lax.bitcast_convert_type(out_u32, jnp.bfloat16)
    return out_bf.astype(jnp.float32).reshape(B, P, D)


# speedup vs baseline: 1.7037x; 1.7037x over previous
import functools

import jax
import jax.numpy as jnp
from jax import lax
from jax.experimental import pallas as pl
from jax.experimental.pallas import tpu as pltpu
from jax.experimental.pallas import tpu_sc as plsc

B, P, D, K = 2, 4096, 768, 16
NC, NS, L = 2, 16, 16
NW = NC * NS
ROWS = B * P
RPW = ROWS // NW
DW = D // 2
DV = DW // L
G = 8
NBUF = 2
NG = RPW // G
DU = 4
PR = 32
NPC = RPW // PR

_mesh = plsc.VectorSubcoreMesh(
    core_axis_name="c", subcore_axis_name="s", num_cores=NC, num_subcores=NS
)
_params = pltpu.CompilerParams(needs_layout_passes=False)


def _rtne_bf16_bits(u):
    return (u + 0x7FFF + ((u >> 16) & 1)) >> 16


@functools.partial(
    pl.kernel,
    out_type=jax.ShapeDtypeStruct((ROWS, DW), jnp.uint32),
    mesh=_mesh,
    compiler_params=_params,
    scratch_types=[
        [pltpu.VMEM((PR, D), jnp.float32) for _ in range(2)],
        [pltpu.VMEM((PR, DW), jnp.uint32) for _ in range(2)],
        [pltpu.SemaphoreType.DMA for _ in range(2)],
    ],
)
def _pack_rows(x_hbm, xp_hbm, in_bufs, out_bufs, sems):
    wid = lax.axis_index("s") * NC + lax.axis_index("c")
    wstart = wid * RPW

    def issue(c, slot):
        base = wstart + c * PR
        pltpu.make_async_copy(
            x_hbm.at[pl.ds(base, PR)], in_bufs[slot], sems[slot]
        ).start()

    def convert(c, slot):
        base = wstart + c * PR
        pltpu.make_async_copy(
            x_hbm.at[pl.ds(base, PR)], in_bufs[slot], sems[slot]
        ).wait()

        def rbody(r, carry, slot=slot):
            for j in range(DV):
                sl = pl.ds(j * L, L)
                sh = pl.ds(DW + j * L, L)
                a = plsc.bitcast(in_bufs[slot][r, sl], jnp.uint32)
                b = plsc.bitcast(in_bufs[slot][r, sh], jnp.uint32)
                lo = _rtne_bf16_bits(a)
                hi = _rtne_bf16_bits(b)
                out_bufs[slot][r, sl] = lo | (hi << 16)
            return carry

        lax.fori_loop(0, PR, rbody, 0)
        pltpu.sync_copy(out_bufs[slot], xp_hbm.at[pl.ds(base, PR)])

    issue(0, 0)

    def outer(cc, carry):
        for bslot in range(2):
            c = cc * 2 + bslot

            @pl.when(c + 1 < NPC)
            def _issue_next():
                issue(c + 1, (bslot + 1) % 2)

            convert(c, bslot)
        return carry

    lax.fori_loop(0, NPC // 2, outer, 0)


@functools.partial(
    pl.kernel,
    out_type=jax.ShapeDtypeStruct((ROWS, D), jnp.float32),
    mesh=_mesh,
    compiler_params=_params,
    scratch_types=[
        [pltpu.VMEM((G * K,), jnp.int32) for _ in range(NBUF)],
        [pltpu.VMEM((G * K,), jnp.float32) for _ in range(NBUF)],
        [pltpu.VMEM((G * K, DW), jnp.uint32) for _ in range(NBUF)],
        pltpu.VMEM((G, D), jnp.float32),
        [pltpu.SemaphoreType.DMA for _ in range(NBUF)],
    ],
)
def _wormhole_gather(xp_hbm, routes_hbm, weights_hbm, out_hbm,
                     idx_bufs, w_bufs, row_bufs, out_v, sems):
    wid = lax.axis_index("s") * NC + lax.axis_index("c")
    wstart = wid * RPW
    boff = (wid // (NW // B)) * P
    off_splat = jnp.broadcast_to(boff, (L,)).astype(jnp.int32)

    def issue(c, slot):
        base_k = pl.multiple_of((wstart + c * G) * K, G * K)
        pltpu.sync_copy(routes_hbm.at[pl.ds(base_k, G * K)], idx_bufs[slot])
        for i in range(G * K // L):
            sl = pl.ds(i * L, L)
            idx_bufs[slot][sl] = idx_bufs[slot][sl] + off_splat
        pltpu.sync_copy(weights_hbm.at[pl.ds(base_k, G * K)], w_bufs[slot])
        pltpu.make_async_copy(
            xp_hbm.at[idx_bufs[slot]], row_bufs[slot], sems[slot]
        ).start()

    def compute(c, slot):
        base = wstart + c * G
        pltpu.make_async_copy(
            xp_hbm.at[idx_bufs[slot]], row_bufs[slot], sems[slot]
        ).wait()
        rows = row_bufs[slot]
        for g in range(G):
            wv = w_bufs[slot][pl.ds(g * K, K)]
            r = _rtne_bf16_bits(plsc.bitcast(wv, jnp.uint32))
            pk = r | (r << 16)
            ws = [
                plsc.bitcast(jnp.broadcast_to(pk[k], (L,)), jnp.bfloat16)
                for k in range(K)
            ]

            def dbody(d, carry, g=g, ws=ws):
                for u_ in range(DU):
                    j = d * DU + u_
                    sl = pl.ds(j * L, L)
                    ld = lambda k: plsc.bitcast(rows[g * K + k, sl],
                                                jnp.bfloat16)
                    acc = [ld(a) * ws[a] for a in range(4)]
                    for k in range(4, K):
                        acc[k % 4] = acc[k % 4] + ld(k) * ws[k]
                    s = (acc[0] + acc[1]) + (acc[2] + acc[3])
                    su = plsc.bitcast(s, jnp.uint32)
                    out_v[g, sl] = plsc.bitcast(su << 16, jnp.float32)
                    out_v[g, pl.ds(DW + j * L, L)] = plsc.bitcast(
                        su & jnp.uint32(0xFFFF0000), jnp.float32)
                return carry

            lax.fori_loop(0, DV // DU, dbody, 0)
        pltpu.sync_copy(out_v, out_hbm.at[pl.ds(base, G)])

    issue(0, 0)

    def outer(cc, carry):
        for bslot in range(NBUF):
            c = cc * NBUF + bslot

            @pl.when(c + 1 < NG)
            def _issue_next():
                issue(c + 1, (bslot + 1) % NBUF)

            compute(c, bslot)
        return carry

    lax.fori_loop(0, NG // NBUF, outer, 0)


def kernel(x, routes, weights):
    x_flat = x.reshape(ROWS, D)
    r_flat = routes.astype(jnp.int32).reshape(ROWS * K)
    w_flat = weights.reshape(ROWS * K)
    xp = _pack_rows(x_flat)
    out = _wormhole_gather(xp, r_flat, w_flat)
    return out.reshape(B, P, D)


# speedup vs baseline: 1.9111x; 1.1218x over previous
import functools

import jax
import jax.numpy as jnp
from jax import lax
from jax.experimental import pallas as pl
from jax.experimental.pallas import tpu as pltpu
from jax.experimental.pallas import tpu_sc as plsc

B, P, D, K = 2, 4096, 768, 16
NC, NS, L = 2, 16, 16
NW = NC * NS
ROWS = B * P
RPW = ROWS // NW
DW = D // 2
DV = DW // L
G = 8
NBUF = 2
NG = RPW // G
DU = 4
PR = 32
NPC = RPW // PR

_mesh = plsc.VectorSubcoreMesh(
    core_axis_name="c", subcore_axis_name="s", num_cores=NC, num_subcores=NS
)
_params = pltpu.CompilerParams(needs_layout_passes=False)


def _rtne_bf16_bits(u):
    return (u + 0x7FFF + ((u >> 16) & 1)) >> 16


@functools.partial(
    pl.kernel,
    out_type=jax.ShapeDtypeStruct((ROWS, DW), jnp.uint32),
    mesh=_mesh,
    compiler_params=_params,
    scratch_types=[
        [pltpu.VMEM((PR, D), jnp.float32) for _ in range(2)],
        [pltpu.VMEM((PR, DW), jnp.uint32) for _ in range(2)],
        [pltpu.SemaphoreType.DMA for _ in range(2)],
    ],
)
def _pack_rows(x_hbm, xp_hbm, in_bufs, out_bufs, sems):
    wid = lax.axis_index("s") * NC + lax.axis_index("c")
    wstart = wid * RPW

    def issue(c, slot):
        base = wstart + c * PR
        pltpu.make_async_copy(
            x_hbm.at[pl.ds(base, PR)], in_bufs[slot], sems[slot]
        ).start()

    def convert(c, slot):
        base = wstart + c * PR
        pltpu.make_async_copy(
            x_hbm.at[pl.ds(base, PR)], in_bufs[slot], sems[slot]
        ).wait()

        def rbody(r, carry, slot=slot):
            for j in range(DV):
                sl = pl.ds(j * L, L)
                sh = pl.ds(DW + j * L, L)
                a = plsc.bitcast(in_bufs[slot][r, sl], jnp.uint32)
                b = plsc.bitcast(in_bufs[slot][r, sh], jnp.uint32)
                lo = _rtne_bf16_bits(a)
                hi = _rtne_bf16_bits(b)
                out_bufs[slot][r, sl] = lo | (hi << 16)
            return carry

        lax.fori_loop(0, PR, rbody, 0)
        pltpu.sync_copy(out_bufs[slot], xp_hbm.at[pl.ds(base, PR)])

    issue(0, 0)

    def outer(cc, carry):
        for bslot in range(2):
            c = cc * 2 + bslot

            @pl.when(c + 1 < NPC)
            def _issue_next():
                issue(c + 1, (bslot + 1) % 2)

            convert(c, bslot)
        return carry

    lax.fori_loop(0, NPC // 2, outer, 0)


@functools.partial(
    pl.kernel,
    out_type=jax.ShapeDtypeStruct((ROWS, D), jnp.float32),
    mesh=_mesh,
    compiler_params=_params,
    scratch_types=[
        pltpu.VMEM((RPW * K,), jnp.int32),
        pltpu.VMEM((RPW * K,), jnp.float32),
        [pltpu.VMEM((G * K, DW), jnp.uint32) for _ in range(NBUF)],
        pltpu.VMEM((G, D), jnp.float32),
        [pltpu.SemaphoreType.DMA for _ in range(NBUF)],
        pltpu.SemaphoreType.DMA,
    ],
)
def _wormhole_gather(xp_hbm, routes_hbm, weights_hbm, out_hbm,
                     idx_all, w_all, row_bufs, out_v, sems, psem):
    wid = lax.axis_index("s") * NC + lax.axis_index("c")
    wstart = wid * RPW
    boff = (wid // (NW // B)) * P
    off_splat = jnp.broadcast_to(boff, (L,)).astype(jnp.int32)

    base_all = pl.multiple_of(wstart * K, RPW * K)
    pltpu.make_async_copy(
        weights_hbm.at[pl.ds(base_all, RPW * K)], w_all, psem
    ).start()
    pltpu.sync_copy(routes_hbm.at[pl.ds(base_all, RPW * K)], idx_all)

    def obody(i, carry):
        sl = pl.ds(i * L, L)
        idx_all[sl] = idx_all[sl] + off_splat
        return carry

    lax.fori_loop(0, RPW * K // L, obody, 0)
    pltpu.make_async_copy(
        weights_hbm.at[pl.ds(base_all, RPW * K)], w_all, psem
    ).wait()

    def issue(c, slot):
        pltpu.make_async_copy(
            xp_hbm.at[idx_all.at[pl.ds(c * (G * K), G * K)]],
            row_bufs[slot], sems[slot],
        ).start()

    def compute(c, slot):
        base = wstart + c * G
        pltpu.make_async_copy(
            xp_hbm.at[idx_all.at[pl.ds(c * (G * K), G * K)]],
            row_bufs[slot], sems[slot],
        ).wait()
        rows = row_bufs[slot]
        for g in range(G):
            wv = w_all[pl.ds((c * G + g) * K, K)]
            r = _rtne_bf16_bits(plsc.bitcast(wv, jnp.uint32))
            pk = r | (r << 16)
            ws = [
                plsc.bitcast(jnp.broadcast_to(pk[k], (L,)), jnp.bfloat16)
                for k in range(K)
            ]

            def dbody(d, carry, g=g, ws=ws):
                for u_ in range(DU):
                    j = d * DU + u_
                    sl = pl.ds(j * L, L)
                    ld = lambda k: plsc.bitcast(rows[g * K + k, sl],
                                                jnp.bfloat16)
                    acc = [ld(a) * ws[a] for a in range(4)]
                    for k in range(4, K):
                        acc[k % 4] = acc[k % 4] + ld(k) * ws[k]
                    s = (acc[0] + acc[1]) + (acc[2] + acc[3])
                    su = plsc.bitcast(s, jnp.uint32)
                    out_v[g, sl] = plsc.bitcast(su << 16, jnp.float32)
                    out_v[g, pl.ds(DW + j * L, L)] = plsc.bitcast(
                        su & jnp.uint32(0xFFFF0000), jnp.float32)
                return carry

            lax.fori_loop(0, DV // DU, dbody, 0)
        pltpu.sync_copy(out_v, out_hbm.at[pl.ds(base, G)])

    issue(0, 0)

    def outer(cc, carry):
        for bslot in range(NBUF):
            c = cc * NBUF + bslot

            @pl.when(c + 1 < NG)
            def _issue_next():
                issue(c + 1, (bslot + 1) % NBUF)

            compute(c, bslot)
        return carry

    lax.fori_loop(0, NG // NBUF, outer, 0)


def kernel(x, routes, weights):
    x_flat = x.reshape(ROWS, D)
    r_flat = routes.astype(jnp.int32).reshape(ROWS * K)
    w_flat = weights.reshape(ROWS * K)
    xp = _pack_rows(x_flat)
    out = _wormhole_gather(xp, r_flat, w_flat)
    return out.reshape(B, P, D)


# speedup vs baseline: 2.9033x; 1.5191x over previous
import functools

import jax
import jax.numpy as jnp
from jax import lax
from jax.experimental import pallas as pl
from jax.experimental.pallas import tpu as pltpu
from jax.experimental.pallas import tpu_sc as plsc

B, P, D, K = 2, 4096, 768, 16
NC, NS, L = 2, 16, 16
ROWS = B * P
RPW = P // NS
DW = D // 2
DV = DW // L
G = 8
NBUF = 2
NG = RPW // G
DU = 4
PR = 32
NPC = RPW // PR

_mesh = plsc.VectorSubcoreMesh(
    core_axis_name="c", subcore_axis_name="s", num_cores=NC, num_subcores=NS
)
_params = pltpu.CompilerParams(needs_layout_passes=False)


def _rtne_bf16_bits(u):
    return (u + 0x7FFF + ((u >> 16) & 1)) >> 16


@functools.partial(
    pl.kernel,
    out_type=(
        jax.ShapeDtypeStruct((ROWS, D), jnp.float32),
        jax.ShapeDtypeStruct((ROWS, DW), jnp.uint32),
    ),
    mesh=_mesh,
    compiler_params=_params,
    scratch_types=[
        pltpu.VMEM((RPW * K,), jnp.int32),
        pltpu.VMEM((RPW * K,), jnp.float32),
        pltpu.SemaphoreType.DMA,
        pltpu.SemaphoreType.DMA,
    ],
)
def _wormhole_gather(x_hbm, routes_hbm, weights_hbm, out_hbm, xp_hbm,
                     idx_all, w_all, isem, wsem):
    c = lax.axis_index("c")
    s = lax.axis_index("s")
    xbase = c * P + s * RPW

    base_all = pl.multiple_of(xbase * K, RPW * K)
    pltpu.make_async_copy(
        routes_hbm.at[pl.ds(base_all, RPW * K)], idx_all, isem
    ).start()
    pltpu.make_async_copy(
        weights_hbm.at[pl.ds(base_all, RPW * K)], w_all, wsem
    ).start()

    def pack_phase(in_bufs, pk_bufs, psems):
        def pk_issue(pc, slot):
            pltpu.make_async_copy(
                x_hbm.at[pl.ds(xbase + pc * PR, PR)], in_bufs[slot],
                psems[slot],
            ).start()

        def pk_convert(pc, slot):
            pltpu.make_async_copy(
                x_hbm.at[pl.ds(xbase + pc * PR, PR)], in_bufs[slot],
                psems[slot],
            ).wait()

            def rbody(r, carry, slot=slot):
                for j in range(DV):
                    a = plsc.bitcast(in_bufs[slot][r, pl.ds(j * L, L)],
                                     jnp.uint32)
                    b = plsc.bitcast(
                        in_bufs[slot][r, pl.ds(DW + j * L, L)], jnp.uint32)
                    lo = _rtne_bf16_bits(a)
                    hi = _rtne_bf16_bits(b)
                    pk_bufs[slot][r, pl.ds(j * L, L)] = lo | (hi << 16)
                return carry

            lax.fori_loop(0, PR, rbody, 0)
            pltpu.sync_copy(pk_bufs[slot],
                            xp_hbm.at[pl.ds(xbase + pc * PR, PR)])

        pk_issue(0, 0)

        def pk_outer(cc, carry):
            for bslot in range(2):
                pc = cc * 2 + bslot

                @pl.when(pc + 1 < NPC)
                def _issue_next():
                    pk_issue(pc + 1, (bslot + 1) % 2)

                pk_convert(pc, bslot)
            return carry

        lax.fori_loop(0, NPC // 2, pk_outer, 0)

    pl.run_scoped(
        pack_phase,
        [pltpu.VMEM((PR, D), jnp.float32) for _ in range(2)],
        [pltpu.VMEM((PR, DW), jnp.uint32) for _ in range(2)],
        [pltpu.SemaphoreType.DMA for _ in range(2)],
    )

    plsc.subcore_barrier()
    pltpu.make_async_copy(
        routes_hbm.at[pl.ds(base_all, RPW * K)], idx_all, isem
    ).wait()
    pltpu.make_async_copy(
        weights_hbm.at[pl.ds(base_all, RPW * K)], w_all, wsem
    ).wait()
    boff = c * P
    off_splat = jnp.broadcast_to(boff, (L,)).astype(jnp.int32)

    def obody(i, carry):
        sl = pl.ds(i * L, L)
        idx_all[sl] = idx_all[sl] + off_splat
        return carry

    lax.fori_loop(0, RPW * K // L, obody, 0)

    def gather_phase(row_bufs, out_v, sems):
        def issue(gc, slot):
            pltpu.make_async_copy(
                xp_hbm.at[idx_all.at[pl.ds(gc * (G * K), G * K)]],
                row_bufs[slot], sems[slot],
            ).start()

        def compute(gc, slot):
            pltpu.make_async_copy(
                xp_hbm.at[idx_all.at[pl.ds(gc * (G * K), G * K)]],
                row_bufs[slot], sems[slot],
            ).wait()
            rows = row_bufs[slot]
            for g in range(G):
                wv = w_all[pl.ds((gc * G + g) * K, K)]
                r = _rtne_bf16_bits(plsc.bitcast(wv, jnp.uint32))
                pk = r | (r << 16)
                ws = [
                    plsc.bitcast(jnp.broadcast_to(pk[k], (L,)),
                                 jnp.bfloat16)
                    for k in range(K)
                ]

                def dbody(d, carry, g=g, ws=ws):
                    for u_ in range(DU):
                        j = d * DU + u_
                        sl = pl.ds(j * L, L)
                        ld = lambda k: plsc.bitcast(rows[g * K + k, sl],
                                                    jnp.bfloat16)
                        acc = [ld(a) * ws[a] for a in range(4)]
                        for k in range(4, K):
                            acc[k % 4] = acc[k % 4] + ld(k) * ws[k]
                        ssum = (acc[0] + acc[1]) + (acc[2] + acc[3])
                        su = plsc.bitcast(ssum, jnp.uint32)
                        out_v[g, sl] = plsc.bitcast(su << 16, jnp.float32)
                        out_v[g, pl.ds(DW + j * L, L)] = plsc.bitcast(
                            su & jnp.uint32(0xFFFF0000), jnp.float32)
                    return carry

                lax.fori_loop(0, DV // DU, dbody, 0)
            pltpu.sync_copy(out_v, out_hbm.at[pl.ds(xbase + gc * G, G)])

        issue(0, 0)

        def outer(cc, carry):
            for bslot in range(NBUF):
                gc = cc * NBUF + bslot

                @pl.when(gc + 1 < NG)
                def _issue_next():
                    issue(gc + 1, (bslot + 1) % NBUF)

                compute(gc, bslot)
            return carry

        lax.fori_loop(0, NG // NBUF, outer, 0)

    pl.run_scoped(
        gather_phase,
        [pltpu.VMEM((G * K, DW), jnp.uint32) for _ in range(NBUF)],
        pltpu.VMEM((G, D), jnp.float32),
        [pltpu.SemaphoreType.DMA for _ in range(NBUF)],
    )


def kernel(x, routes, weights):
    x_flat = x.reshape(ROWS, D)
    r_flat = routes.astype(jnp.int32).reshape(ROWS * K)
    w_flat = weights.reshape(ROWS * K)
    out, _ = _wormhole_gather(x_flat, r_flat, w_flat)
    return out.reshape(B, P, D)
